# baseline (device time: 98624 ns/iter reference)
import functools

import jax
import jax.numpy as jnp
from jax import lax
from jax.experimental import pallas as pl
from jax.experimental.pallas import tpu as pltpu

N_DEV = 32
N_STEPS = 5
DH = 64


def kernel(x, Wq, K_ext, V_ext, Wo):
    B, Sq, E = x.shape
    H_local = Wq.shape[1] // DH
    Skv = K_ext.shape[1]
    rows = B * Sq

    halves = [rows >> (k + 1) for k in range(N_STEPS)]
    slot_off = [sum(halves[:k]) for k in range(N_STEPS)]
    comm_rows = sum(halves)

    def body(x_ref, wq_ref, k_hbm, v_hbm, wo_ref, out_ref,
             comm_ref, k_ref, v_ref, kv_sems, send_sems, recv_sems):
        p = lax.axis_index("i")
        z = p // 8
        j = lax.rem(p, 8)
        y = j // 2
        xx = lax.rem(j + y, 2)

        def logical(x_, y_, z_):
            return 8 * z_ + 2 * y_ + lax.rem(x_ + y_, 2)

        bits = [xx, lax.rem(y, 2), y // 2, lax.rem(z, 2), z // 2]
        partners = [
            logical(1 - xx, y, z),
            logical(xx, y + 1 - 2 * bits[1], z),
            logical(xx, y + 2 - 4 * bits[2], z),
            logical(xx, y, z + 1 - 2 * bits[3]),
            logical(xx, y, z + 2 - 4 * bits[4]),
        ]

        h0 = p * H_local
        copy_k = pltpu.make_async_copy(
            k_hbm.at[:, :, pl.ds(h0, H_local), :], k_ref, kv_sems.at[0])
        copy_v = pltpu.make_async_copy(
            v_hbm.at[:, :, pl.ds(h0, H_local), :], v_ref, kv_sems.at[1])
        copy_k.start()
        copy_v.start()

        xf = x_ref[...].reshape(rows, E).astype(jnp.bfloat16)
        wq = wq_ref[...].astype(jnp.bfloat16)
        q_all = jnp.dot(xf, wq, preferred_element_type=jnp.float32)

        qb = lax.broadcasted_iota(jnp.int32, (Sq, Skv), 0) // 64
        kb = lax.broadcasted_iota(jnp.int32, (Sq, Skv), 1) // 64
        mask = (qb == kb) | ((kb % 4) == (qb % 4))

        copy_k.wait()
        copy_v.wait()

        ctx_rows = []
        for b in range(B):
            ctx_heads = []
            for h in range(H_local):
                q = q_all[b * Sq:(b + 1) * Sq, h * DH:(h + 1) * DH]
                k = k_ref[b, :, h, :].astype(jnp.bfloat16)
                s = jnp.dot(q.astype(jnp.bfloat16), k.T,
                            preferred_element_type=jnp.float32) * 0.125
                s = jnp.where(mask, s, -1e9)
                s = s - jnp.max(s, axis=1, keepdims=True)
                e = jnp.exp(s)
                w = e / jnp.sum(e, axis=1, keepdims=True)
                v = v_ref[b, :, h, :].astype(jnp.bfloat16)
                ctx_heads.append(jnp.dot(w.astype(jnp.bfloat16), v,
                                         preferred_element_type=jnp.float32))
            ctx_rows.append(jnp.concatenate(ctx_heads, axis=1))
        ctx = jnp.concatenate(ctx_rows, axis=0)

        wo = wo_ref[...].astype(jnp.bfloat16)
        out_ref[...] = jnp.dot(ctx.astype(jnp.bfloat16), wo,
                               preferred_element_type=jnp.float32)

        barrier_sem = pltpu.get_barrier_semaphore()
        for k in range(N_STEPS):
            pl.semaphore_signal(barrier_sem, inc=1, device_id=(partners[k],),
                                device_id_type=pl.DeviceIdType.MESH)
        pl.semaphore_wait(barrier_sem, N_STEPS)

        base = jnp.int32(0)
        for k in range(N_STEPS):
            half = halves[k]
            send_start = base + (1 - bits[k]) * half
            keep_start = base + bits[k] * half
            rdma = pltpu.make_async_remote_copy(
                src_ref=out_ref.at[pl.ds(send_start, half)],
                dst_ref=comm_ref.at[pl.ds(slot_off[k], half)],
                send_sem=send_sems.at[k],
                recv_sem=recv_sems.at[k],
                device_id=(partners[k],),
                device_id_type=pl.DeviceIdType.MESH,
            )
            rdma.start()
            rdma.wait()
            kr = pl.ds(keep_start, half)
            out_ref[kr, :] = (out_ref[kr, :]
                              + comm_ref[pl.ds(slot_off[k], half), :])
            base = keep_start

        for k in reversed(range(N_STEPS)):
            size = halves[k]
            rdma = pltpu.make_async_remote_copy(
                src_ref=out_ref.at[pl.ds(base, size)],
                dst_ref=out_ref.at[pl.ds(base, size)],
                send_sem=send_sems.at[N_STEPS + k],
                recv_sem=recv_sems.at[N_STEPS + k],
                device_id=(partners[k],),
                device_id_type=pl.DeviceIdType.MESH,
            )
            rdma.start()
            rdma.wait()
            base = base - bits[k] * size

        @functools.partial(pl.run_scoped,
                           second_barrier=pltpu.SemaphoreType.REGULAR)
        def _(second_barrier):
            for k in range(N_STEPS):
                pl.semaphore_signal(second_barrier, inc=1,
                                    device_id=(partners[k],),
                                    device_id_type=pl.DeviceIdType.MESH)
            pl.semaphore_wait(second_barrier, N_STEPS)

    out = pl.pallas_call(
        body,
        out_shape=jax.ShapeDtypeStruct((rows, E), jnp.float32),
        in_specs=[
            pl.BlockSpec(memory_space=pltpu.VMEM),
            pl.BlockSpec(memory_space=pltpu.VMEM),
            pl.BlockSpec(memory_space=pl.ANY),
            pl.BlockSpec(memory_space=pl.ANY),
            pl.BlockSpec(memory_space=pltpu.VMEM),
        ],
        out_specs=pl.BlockSpec(memory_space=pltpu.VMEM),
        scratch_shapes=[
            pltpu.VMEM((comm_rows, E), jnp.float32),
            pltpu.VMEM((B, Skv, H_local, DH), jnp.float32),
            pltpu.VMEM((B, Skv, H_local, DH), jnp.float32),
            pltpu.SemaphoreType.DMA((2,)),
            pltpu.SemaphoreType.DMA((2 * N_STEPS,)),
            pltpu.SemaphoreType.DMA((2 * N_STEPS,)),
        ],
        compiler_params=pltpu.CompilerParams(collective_id=0),
    )(x, Wq, K_ext, V_ext, Wo)
    return out.reshape(B, Sq, E)


# device time: 98454 ns/iter; 1.0017x vs baseline; 1.0017x over previous
import functools

import jax
import jax.numpy as jnp
from jax import lax
from jax.experimental import pallas as pl
from jax.experimental.pallas import tpu as pltpu

N_DEV = 32
N_STEPS = 5
DH = 64


def kernel(x, Wq, K_ext, V_ext, Wo):
    B, Sq, E = x.shape
    H_local = Wq.shape[1] // DH
    Skv = K_ext.shape[1]
    rows = B * Sq

    halves = [rows >> (k + 1) for k in range(N_STEPS)]
    slot_off = [sum(halves[:k]) for k in range(N_STEPS)]
    comm_rows = sum(halves)

    def body(x_ref, wq_ref, k_hbm, v_hbm, wo_ref, out_ref,
             comm_ref, k_ref, v_ref, kv_sems, send_sems, recv_sems):
        p = lax.axis_index("i")
        z = p // 8
        j = lax.rem(p, 8)
        y = j // 2
        xx = lax.rem(j + y, 2)

        def logical(x_, y_, z_):
            return 8 * z_ + 2 * y_ + lax.rem(x_ + y_, 2)

        bits = [xx, lax.rem(y, 2), y // 2, lax.rem(z, 2), z // 2]
        partners = [
            logical(1 - xx, y, z),
            logical(xx, y + 1 - 2 * bits[1], z),
            logical(xx, y + 2 - 4 * bits[2], z),
            logical(xx, y, z + 1 - 2 * bits[3]),
            logical(xx, y, z + 2 - 4 * bits[4]),
        ]

        h0 = p * H_local
        n_seq_split = 4
        seq_q = Skv // n_seq_split
        kv_copies = []
        for src, dst, base_sem in ((k_hbm, k_ref, 0), (v_hbm, v_ref, 8)):
            for b in range(B):
                for sq in range(n_seq_split):
                    c = pltpu.make_async_copy(
                        src.at[b, pl.ds(sq * seq_q, seq_q),
                               pl.ds(h0, H_local), :],
                        dst.at[b, pl.ds(sq * seq_q, seq_q), :, :],
                        kv_sems.at[base_sem + b * n_seq_split + sq],
                    )
                    c.start()
                    kv_copies.append(c)

        xf = x_ref[...].reshape(rows, E).astype(jnp.bfloat16)
        wq = wq_ref[...].astype(jnp.bfloat16)
        q_all = jnp.dot(xf, wq, preferred_element_type=jnp.float32)

        qb = lax.broadcasted_iota(jnp.int32, (Sq, Skv), 0) // 64
        kb = lax.broadcasted_iota(jnp.int32, (Sq, Skv), 1) // 64
        mask = (qb == kb) | ((kb % 4) == (qb % 4))

        for c in kv_copies:
            c.wait()

        ctx_rows = []
        for b in range(B):
            ctx_heads = []
            for h in range(H_local):
                q = q_all[b * Sq:(b + 1) * Sq, h * DH:(h + 1) * DH]
                k = k_ref[b, :, h, :].astype(jnp.bfloat16)
                s = jnp.dot(q.astype(jnp.bfloat16), k.T,
                            preferred_element_type=jnp.float32) * 0.125
                s = jnp.where(mask, s, -1e9)
                s = s - jnp.max(s, axis=1, keepdims=True)
                e = jnp.exp(s)
                w = e / jnp.sum(e, axis=1, keepdims=True)
                v = v_ref[b, :, h, :].astype(jnp.bfloat16)
                ctx_heads.append(jnp.dot(w.astype(jnp.bfloat16), v,
                                         preferred_element_type=jnp.float32))
            ctx_rows.append(jnp.concatenate(ctx_heads, axis=1))
        ctx = jnp.concatenate(ctx_rows, axis=0)

        wo = wo_ref[...].astype(jnp.bfloat16)
        out_ref[...] = jnp.dot(ctx.astype(jnp.bfloat16), wo,
                               preferred_element_type=jnp.float32)

        barrier_sem = pltpu.get_barrier_semaphore()
        for k in range(N_STEPS):
            pl.semaphore_signal(barrier_sem, inc=1, device_id=(partners[k],),
                                device_id_type=pl.DeviceIdType.MESH)
        pl.semaphore_wait(barrier_sem, N_STEPS)

        base = jnp.int32(0)
        for k in range(N_STEPS):
            half = halves[k]
            send_start = base + (1 - bits[k]) * half
            keep_start = base + bits[k] * half
            rdma = pltpu.make_async_remote_copy(
                src_ref=out_ref.at[pl.ds(send_start, half)],
                dst_ref=comm_ref.at[pl.ds(slot_off[k], half)],
                send_sem=send_sems.at[k],
                recv_sem=recv_sems.at[k],
                device_id=(partners[k],),
                device_id_type=pl.DeviceIdType.MESH,
            )
            rdma.start()
            rdma.wait()
            kr = pl.ds(keep_start, half)
            out_ref[kr, :] = (out_ref[kr, :]
                              + comm_ref[pl.ds(slot_off[k], half), :])
            base = keep_start

        for k in reversed(range(N_STEPS)):
            size = halves[k]
            rdma = pltpu.make_async_remote_copy(
                src_ref=out_ref.at[pl.ds(base, size)],
                dst_ref=out_ref.at[pl.ds(base, size)],
                send_sem=send_sems.at[N_STEPS + k],
                recv_sem=recv_sems.at[N_STEPS + k],
                device_id=(partners[k],),
                device_id_type=pl.DeviceIdType.MESH,
            )
            rdma.start()
            rdma.wait()
            base = base - bits[k] * size

        @functools.partial(pl.run_scoped,
                           second_barrier=pltpu.SemaphoreType.REGULAR)
        def _(second_barrier):
            for k in range(N_STEPS):
                pl.semaphore_signal(second_barrier, inc=1,
                                    device_id=(partners[k],),
                                    device_id_type=pl.DeviceIdType.MESH)
            pl.semaphore_wait(second_barrier, N_STEPS)

    out = pl.pallas_call(
        body,
        out_shape=jax.ShapeDtypeStruct((rows, E), jnp.float32),
        in_specs=[
            pl.BlockSpec(memory_space=pltpu.VMEM),
            pl.BlockSpec(memory_space=pltpu.VMEM),
            pl.BlockSpec(memory_space=pl.ANY),
            pl.BlockSpec(memory_space=pl.ANY),
            pl.BlockSpec(memory_space=pltpu.VMEM),
        ],
        out_specs=pl.BlockSpec(memory_space=pltpu.VMEM),
        scratch_shapes=[
            pltpu.VMEM((comm_rows, E), jnp.float32),
            pltpu.VMEM((B, Skv, H_local, DH), jnp.float32),
            pltpu.VMEM((B, Skv, H_local, DH), jnp.float32),
            pltpu.SemaphoreType.DMA((16,)),
            pltpu.SemaphoreType.DMA((2 * N_STEPS,)),
            pltpu.SemaphoreType.DMA((2 * N_STEPS,)),
        ],
        compiler_params=pltpu.CompilerParams(collective_id=0),
    )(x, Wq, K_ext, V_ext, Wo)
    return out.reshape(B, Sq, E)


# device time: 94622 ns/iter; 1.0423x vs baseline; 1.0405x over previous
import functools
import os

import jax
import jax.numpy as jnp
from jax import lax
from jax.experimental import pallas as pl
from jax.experimental.pallas import tpu as pltpu

N_DEV = 32
N_STEPS = 5
DH = 64

_SKIP_COMM = os.environ.get("ABLATE_SKIP_COMM") == "1"
_SKIP_KV = os.environ.get("ABLATE_SKIP_KV") == "1"
_SKIP_ATTN = os.environ.get("ABLATE_SKIP_ATTN") == "1"


def kernel(x, Wq, K_ext, V_ext, Wo):
    B, Sq, E = x.shape
    H_local = Wq.shape[1] // DH
    Skv = K_ext.shape[1]
    H_glob = K_ext.shape[2]
    rows = B * Sq

    K2 = jnp.swapaxes(K_ext, 2, 3).reshape(B, Skv, DH * H_glob)
    V2 = jnp.swapaxes(V_ext, 2, 3).reshape(B, Skv, DH * H_glob)

    halves = [rows >> (k + 1) for k in range(N_STEPS)]
    slot_off = [sum(halves[:k]) for k in range(N_STEPS)]
    comm_rows = sum(halves)

    def _comm(p, bits, partners, out_ref, comm_ref, send_sems, recv_sems):
        if _SKIP_COMM:
            return

        barrier_sem = pltpu.get_barrier_semaphore()
        for k in range(N_STEPS):
            pl.semaphore_signal(barrier_sem, inc=1, device_id=(partners[k],),
                                device_id_type=pl.DeviceIdType.MESH)
        pl.semaphore_wait(barrier_sem, N_STEPS)

        base = jnp.int32(0)
        for k in range(N_STEPS):
            half = halves[k]
            send_start = base + (1 - bits[k]) * half
            keep_start = base + bits[k] * half
            rdma = pltpu.make_async_remote_copy(
                src_ref=out_ref.at[pl.ds(send_start, half)],
                dst_ref=comm_ref.at[pl.ds(slot_off[k], half)],
                send_sem=send_sems.at[k],
                recv_sem=recv_sems.at[k],
                device_id=(partners[k],),
                device_id_type=pl.DeviceIdType.MESH,
            )
            rdma.start()
            rdma.wait()
            kr = pl.ds(keep_start, half)
            out_ref[kr, :] = (out_ref[kr, :]
                              + comm_ref[pl.ds(slot_off[k], half), :])
            base = keep_start

        for k in reversed(range(N_STEPS)):
            size = halves[k]
            rdma = pltpu.make_async_remote_copy(
                src_ref=out_ref.at[pl.ds(base, size)],
                dst_ref=out_ref.at[pl.ds(base, size)],
                send_sem=send_sems.at[N_STEPS + k],
                recv_sem=recv_sems.at[N_STEPS + k],
                device_id=(partners[k],),
                device_id_type=pl.DeviceIdType.MESH,
            )
            rdma.start()
            rdma.wait()
            base = base - bits[k] * size

        @functools.partial(pl.run_scoped,
                           second_barrier=pltpu.SemaphoreType.REGULAR)
        def _(second_barrier):
            for k in range(N_STEPS):
                pl.semaphore_signal(second_barrier, inc=1,
                                    device_id=(partners[k],),
                                    device_id_type=pl.DeviceIdType.MESH)
            pl.semaphore_wait(second_barrier, N_STEPS)

    def body(x_ref, wq_ref, k_hbm, v_hbm, wo_ref, out_ref,
             comm_ref, buf_ref, k_loc, v_loc, kv_sems, send_sems, recv_sems):
        p = lax.axis_index("i")
        z = p // 8
        j = lax.rem(p, 8)
        y = j // 2
        xx = lax.rem(j + y, 2)

        def logical(x_, y_, z_):
            return 8 * z_ + 2 * y_ + lax.rem(x_ + y_, 2)

        bits = [xx, lax.rem(y, 2), y // 2, lax.rem(z, 2), z // 2]
        partners = [
            logical(1 - xx, y, z),
            logical(xx, y + 1 - 2 * bits[1], z),
            logical(xx, y + 2 - 4 * bits[2], z),
            logical(xx, y, z + 1 - 2 * bits[3]),
            logical(xx, y, z + 2 - 4 * bits[4]),
        ]

        h0 = p * H_local
        H_glob = k_hbm.shape[2] // DH
        gcols = DH * H_glob
        chunk = Skv // 2
        n_chunks = 2 * B * 2

        def chunk_src(t):
            src = k_hbm if t < 4 else v_hbm
            b, sh = (t % 4) // 2, t % 2
            return src.at[b, pl.ds(sh * chunk, chunk), :]

        kv_copies = {}
        if not _SKIP_KV:
            for t in (0, 1):
                kv_copies[t] = pltpu.make_async_copy(
                    chunk_src(t), buf_ref.at[t % 2], kv_sems.at[t % 2])
                kv_copies[t].start()

        xf = x_ref[...].reshape(rows, E).astype(jnp.bfloat16)
        wq = wq_ref[...].astype(jnp.bfloat16)
        q_all = jnp.dot(xf, wq, preferred_element_type=jnp.float32)

        qb = lax.broadcasted_iota(jnp.int32, (Sq, Skv), 0) // 64
        kb = lax.broadcasted_iota(jnp.int32, (Sq, Skv), 1) // 64
        mask = (qb == kb) | ((kb % 4) == (qb % 4))

        if _SKIP_ATTN:
            out_ref[...] = jnp.dot(xf, xf, preferred_element_type=jnp.float32)
            _comm(p, bits, partners, out_ref, comm_ref, send_sems, recv_sems)
            return

        gi = lax.broadcasted_iota(jnp.int32, (gcols, H_local * DH), 0)
        ci = lax.broadcasted_iota(jnp.int32, (gcols, H_local * DH), 1)
        sel = (gi == (ci % DH) * H_glob + h0 + ci // DH)
        S = sel.astype(jnp.bfloat16)

        if not _SKIP_KV:
            for t in range(n_chunks):
                kv_copies[t].wait()
                loc = k_loc if t < 4 else v_loc
                b, sh = (t % 4) // 2, t % 2
                ext = jnp.dot(buf_ref[t % 2].astype(jnp.bfloat16), S,
                              preferred_element_type=jnp.float32)
                loc[b, pl.ds(sh * chunk, chunk), :] = ext.astype(jnp.bfloat16)
                if t + 2 < n_chunks:
                    kv_copies[t + 2] = pltpu.make_async_copy(
                        chunk_src(t + 2), buf_ref.at[t % 2],
                        kv_sems.at[t % 2])
                    kv_copies[t + 2].start()

        ctx_rows = []
        for b in range(B):
            ctx_heads = []
            for h in range(H_local):
                q = q_all[b * Sq:(b + 1) * Sq, h * DH:(h + 1) * DH]
                k = k_loc[b, :, h * DH:(h + 1) * DH]
                s = jnp.dot(q.astype(jnp.bfloat16), k.T,
                            preferred_element_type=jnp.float32) * 0.125
                s = jnp.where(mask, s, -1e9)
                s = s - jnp.max(s, axis=1, keepdims=True)
                e = jnp.exp(s)
                w = e / jnp.sum(e, axis=1, keepdims=True)
                v = v_loc[b, :, h * DH:(h + 1) * DH]
                ctx_heads.append(jnp.dot(w.astype(jnp.bfloat16), v,
                                         preferred_element_type=jnp.float32))
            ctx_rows.append(jnp.concatenate(ctx_heads, axis=1))
        ctx = jnp.concatenate(ctx_rows, axis=0)

        wo = wo_ref[...].astype(jnp.bfloat16)
        out_ref[...] = jnp.dot(ctx.astype(jnp.bfloat16), wo,
                               preferred_element_type=jnp.float32)

        _comm(p, bits, partners, out_ref, comm_ref, send_sems, recv_sems)

    out = pl.pallas_call(
        body,
        out_shape=jax.ShapeDtypeStruct((rows, E), jnp.float32),
        in_specs=[
            pl.BlockSpec(memory_space=pltpu.VMEM),
            pl.BlockSpec(memory_space=pltpu.VMEM),
            pl.BlockSpec(memory_space=pl.ANY),
            pl.BlockSpec(memory_space=pl.ANY),
            pl.BlockSpec(memory_space=pltpu.VMEM),
        ],
        out_specs=pl.BlockSpec(memory_space=pltpu.VMEM),
        scratch_shapes=[
            pltpu.VMEM((comm_rows, E), jnp.float32),
            pltpu.VMEM((2, Skv // 2, DH * H_glob), jnp.float32),
            pltpu.VMEM((B, Skv, H_local * DH), jnp.bfloat16),
            pltpu.VMEM((B, Skv, H_local * DH), jnp.bfloat16),
            pltpu.SemaphoreType.DMA((2,)),
            pltpu.SemaphoreType.DMA((2 * N_STEPS,)),
            pltpu.SemaphoreType.DMA((2 * N_STEPS,)),
        ],
        compiler_params=pltpu.CompilerParams(
            collective_id=None if _SKIP_COMM else 0),
    )(x, Wq, K2, V2, Wo)
    return out.reshape(B, Sq, E)


# device time: 71704 ns/iter; 1.3754x vs baseline; 1.3196x over previous
import functools
import os

import jax
import jax.numpy as jnp
from jax import lax
from jax.experimental import pallas as pl
from jax.experimental.pallas import tpu as pltpu

N_DEV = 32
N_STEPS = 5
DH = 64

_SKIP_COMM = os.environ.get("ABLATE_SKIP_COMM") == "1"
_SKIP_KV = os.environ.get("ABLATE_SKIP_KV") == "1"
_SKIP_ATTN = os.environ.get("ABLATE_SKIP_ATTN") == "1"


def kernel(x, Wq, K_ext, V_ext, Wo):
    B, Sq, E = x.shape
    H_local = Wq.shape[1] // DH
    Skv = K_ext.shape[1]
    H_glob = K_ext.shape[2]
    rows = B * Sq

    K2 = jnp.swapaxes(K_ext, 2, 3)
    V2 = jnp.swapaxes(V_ext, 2, 3)

    halves = [rows >> (k + 1) for k in range(N_STEPS)]
    slot_off = [sum(halves[:k]) for k in range(N_STEPS)]
    comm_rows = sum(halves)

    def _comm(p, bits, partners, out_ref, comm_ref, send_sems, recv_sems):
        if _SKIP_COMM:
            return

        barrier_sem = pltpu.get_barrier_semaphore()
        for k in range(N_STEPS):
            pl.semaphore_signal(barrier_sem, inc=1, device_id=(partners[k],),
                                device_id_type=pl.DeviceIdType.MESH)
        pl.semaphore_wait(barrier_sem, N_STEPS)

        base = jnp.int32(0)
        for k in range(N_STEPS):
            half = halves[k]
            send_start = base + (1 - bits[k]) * half
            keep_start = base + bits[k] * half
            rdma = pltpu.make_async_remote_copy(
                src_ref=out_ref.at[pl.ds(send_start, half)],
                dst_ref=comm_ref.at[pl.ds(slot_off[k], half)],
                send_sem=send_sems.at[k],
                recv_sem=recv_sems.at[k],
                device_id=(partners[k],),
                device_id_type=pl.DeviceIdType.MESH,
            )
            rdma.start()
            rdma.wait()
            kr = pl.ds(keep_start, half)
            out_ref[kr, :] = (out_ref[kr, :]
                              + comm_ref[pl.ds(slot_off[k], half), :])
            base = keep_start

        for k in reversed(range(N_STEPS)):
            size = halves[k]
            rdma = pltpu.make_async_remote_copy(
                src_ref=out_ref.at[pl.ds(base, size)],
                dst_ref=out_ref.at[pl.ds(base, size)],
                send_sem=send_sems.at[N_STEPS + k],
                recv_sem=recv_sems.at[N_STEPS + k],
                device_id=(partners[k],),
                device_id_type=pl.DeviceIdType.MESH,
            )
            rdma.start()
            rdma.wait()
            base = base - bits[k] * size

        @functools.partial(pl.run_scoped,
                           second_barrier=pltpu.SemaphoreType.REGULAR)
        def _(second_barrier):
            for k in range(N_STEPS):
                pl.semaphore_signal(second_barrier, inc=1,
                                    device_id=(partners[k],),
                                    device_id_type=pl.DeviceIdType.MESH)
            pl.semaphore_wait(second_barrier, N_STEPS)

    def body(x_ref, wq_ref, k_hbm, v_hbm, wo_ref, out_ref,
             comm_ref, buf_ref, k_loc, v_loc, kv_sems, send_sems, recv_sems):
        p = lax.axis_index("i")
        z = p // 8
        j = lax.rem(p, 8)
        y = j // 2
        xx = lax.rem(j + y, 2)

        def logical(x_, y_, z_):
            return 8 * z_ + 2 * y_ + lax.rem(x_ + y_, 2)

        bits = [xx, lax.rem(y, 2), y // 2, lax.rem(z, 2), z // 2]
        partners = [
            logical(1 - xx, y, z),
            logical(xx, y + 1 - 2 * bits[1], z),
            logical(xx, y + 2 - 4 * bits[2], z),
            logical(xx, y, z + 1 - 2 * bits[3]),
            logical(xx, y, z + 2 - 4 * bits[4]),
        ]

        h0 = p * H_local
        chunk = Skv // 2
        n_chunks = 2 * B * 2

        def chunk_src(t):
            src = k_hbm if t < 4 else v_hbm
            b, sh = (t % 4) // 2, t % 2
            return src.at[b, pl.ds(sh * chunk, chunk), :, :]

        kv_copies = {}
        if not _SKIP_KV:
            for t in (0, 1):
                kv_copies[t] = pltpu.make_async_copy(
                    chunk_src(t), buf_ref.at[t % 2], kv_sems.at[t % 2])
                kv_copies[t].start()

        xf = x_ref[...].reshape(rows, E).astype(jnp.bfloat16)
        wq = wq_ref[...].astype(jnp.bfloat16)
        q_all = jnp.dot(xf, wq, preferred_element_type=jnp.float32)

        qb = lax.broadcasted_iota(jnp.int32, (Sq, Skv), 0) // 64
        kb = lax.broadcasted_iota(jnp.int32, (Sq, Skv), 1) // 64
        mask = (qb == kb) | ((kb % 4) == (qb % 4))

        if _SKIP_ATTN:
            out_ref[...] = jnp.dot(xf, xf, preferred_element_type=jnp.float32)
            _comm(p, bits, partners, out_ref, comm_ref, send_sems, recv_sems)
            return

        hi = lax.broadcasted_iota(jnp.int32, (H_local, H_glob), 0)
        gi = lax.broadcasted_iota(jnp.int32, (H_local, H_glob), 1)
        S4 = (gi == h0 + hi).astype(jnp.bfloat16)

        if not _SKIP_KV:
            for t in range(n_chunks):
                kv_copies[t].wait()
                loc = k_loc if t < 4 else v_loc
                b, sh = (t % 4) // 2, t % 2
                v2d = buf_ref[t % 2].astype(jnp.bfloat16).reshape(
                    chunk * DH, H_glob)
                sl4 = jnp.dot(S4, v2d.T,
                              preferred_element_type=jnp.float32)
                sl3 = sl4.astype(jnp.bfloat16).reshape(H_local, chunk, DH)
                for hl in range(H_local):
                    loc[b, pl.ds(sh * chunk, chunk),
                        hl * DH:(hl + 1) * DH] = sl3[hl]
                if t + 2 < n_chunks:
                    kv_copies[t + 2] = pltpu.make_async_copy(
                        chunk_src(t + 2), buf_ref.at[t % 2],
                        kv_sems.at[t % 2])
                    kv_copies[t + 2].start()

        ctx_rows = []
        for b in range(B):
            ctx_heads = []
            for h in range(H_local):
                q = q_all[b * Sq:(b + 1) * Sq, h * DH:(h + 1) * DH]
                k = k_loc[b, :, h * DH:(h + 1) * DH]
                s = jnp.dot(q.astype(jnp.bfloat16), k.T,
                            preferred_element_type=jnp.float32) * 0.125
                s = jnp.where(mask, s, -1e9)
                s = s - jnp.max(s, axis=1, keepdims=True)
                e = jnp.exp(s)
                w = e / jnp.sum(e, axis=1, keepdims=True)
                v = v_loc[b, :, h * DH:(h + 1) * DH]
                ctx_heads.append(jnp.dot(w.astype(jnp.bfloat16), v,
                                         preferred_element_type=jnp.float32))
            ctx_rows.append(jnp.concatenate(ctx_heads, axis=1))
        ctx = jnp.concatenate(ctx_rows, axis=0)

        wo = wo_ref[...].astype(jnp.bfloat16)
        out_ref[...] = jnp.dot(ctx.astype(jnp.bfloat16), wo,
                               preferred_element_type=jnp.float32)

        _comm(p, bits, partners, out_ref, comm_ref, send_sems, recv_sems)

    out = pl.pallas_call(
        body,
        out_shape=jax.ShapeDtypeStruct((rows, E), jnp.float32),
        in_specs=[
            pl.BlockSpec(memory_space=pltpu.VMEM),
            pl.BlockSpec(memory_space=pltpu.VMEM),
            pl.BlockSpec(memory_space=pl.ANY),
            pl.BlockSpec(memory_space=pl.ANY),
            pl.BlockSpec(memory_space=pltpu.VMEM),
        ],
        out_specs=pl.BlockSpec(memory_space=pltpu.VMEM),
        scratch_shapes=[
            pltpu.VMEM((comm_rows, E), jnp.float32),
            pltpu.VMEM((2, Skv // 2, DH, H_glob), jnp.float32),
            pltpu.VMEM((B, Skv, H_local * DH), jnp.bfloat16),
            pltpu.VMEM((B, Skv, H_local * DH), jnp.bfloat16),
            pltpu.SemaphoreType.DMA((2,)),
            pltpu.SemaphoreType.DMA((2 * N_STEPS,)),
            pltpu.SemaphoreType.DMA((2 * N_STEPS,)),
        ],
        compiler_params=pltpu.CompilerParams(
            collective_id=None if _SKIP_COMM else 0),
    )(x, Wq, K2, V2, Wo)
    return out.reshape(B, Sq, E)


# device time: 56080 ns/iter; 1.7586x vs baseline; 1.2786x over previous
import functools
import os

import jax
import jax.numpy as jnp
from jax import lax
from jax.experimental import pallas as pl
from jax.experimental.pallas import tpu as pltpu

N_DEV = 32
N_STEPS = 5
DH = 64

_SKIP_COMM = os.environ.get("ABLATE_SKIP_COMM") == "1"
_SKIP_KV = os.environ.get("ABLATE_SKIP_KV") == "1"
_SKIP_ATTN = os.environ.get("ABLATE_SKIP_ATTN") == "1"


def kernel(x, Wq, K_ext, V_ext, Wo):
    B, Sq, E = x.shape
    H_local = Wq.shape[1] // DH
    Skv = K_ext.shape[1]
    H_glob = K_ext.shape[2]
    rows = B * Sq

    K2 = jnp.swapaxes(K_ext, 2, 3)
    V2 = jnp.swapaxes(V_ext, 2, 3)

    halves = [rows >> (k + 1) for k in range(N_STEPS)]
    slot_off = [sum(halves[:k]) for k in range(N_STEPS)]
    comm_rows = sum(halves)

    def _comm(p, xx, y, z, out_ref, comm_ref, stage_ref,
              send_sems, recv_sems):
        if _SKIP_COMM:
            return

        def logical(x_, y_, z_):
            return 8 * z_ + 2 * y_ + lax.rem(x_ + y_, 2)

        def qoff(w, quarter):
            return lax.rem(w, 2) * (2 * quarter) + (w // 2) * quarter

        x_partner = logical(1 - xx, y, z)
        y_partners = [logical(xx, lax.rem(y + dy, 4), z) for dy in (1, 2, 3)]
        z_partners = [logical(xx, y, lax.rem(z + dz, 4)) for dz in (1, 2, 3)]
        all_partners = [x_partner] + y_partners + z_partners

        barrier_sem = pltpu.get_barrier_semaphore()
        for nbr in all_partners:
            pl.semaphore_signal(barrier_sem, inc=1, device_id=(nbr,),
                                device_id_type=pl.DeviceIdType.MESH)
        pl.semaphore_wait(barrier_sem, len(all_partners))

        RSX, RSY, RSZ = 0, 256, 448
        AGZ, AGY, AGX = 496, 544, 736

        def exchange(src_starts, sizes, dst_slots, partner_ids, sems):
            rdmas = []
            soff = 0
            for i, (ss, sz) in enumerate(zip(src_starts, sizes)):
                stage_ref[pl.ds(soff, sz), :] = (
                    out_ref[pl.ds(ss, sz), :].astype(jnp.bfloat16))
                rdmas.append(pltpu.make_async_remote_copy(
                    src_ref=stage_ref.at[pl.ds(soff, sz)],
                    dst_ref=comm_ref.at[pl.ds(dst_slots[i], sz)],
                    send_sem=send_sems.at[sems[i]],
                    recv_sem=recv_sems.at[sems[i]],
                    device_id=(partner_ids[i],),
                    device_id_type=pl.DeviceIdType.MESH,
                ))
                soff += sz
            for r in rdmas:
                r.start()
            for r in rdmas:
                r.wait()

        exchange([(1 - xx) * 256], [256], [RSX], [x_partner], [0])
        out_ref[pl.ds(xx * 256, 256), :] = (
            out_ref[pl.ds(xx * 256, 256), :]
            + comm_ref[pl.ds(RSX, 256), :].astype(jnp.float32))
        b0 = xx * 256

        exchange(
            [b0 + qoff(lax.rem(y + dy, 4), 64) for dy in (1, 2, 3)],
            [64] * 3,
            [RSY, RSY + 64, RSY + 128],
            y_partners, [1, 2, 3])
        kr = pl.ds(b0 + qoff(y, 64), 64)
        out_ref[kr, :] = (
            out_ref[kr, :]
            + comm_ref[pl.ds(RSY, 64), :].astype(jnp.float32)
            + comm_ref[pl.ds(RSY + 64, 64), :].astype(jnp.float32)
            + comm_ref[pl.ds(RSY + 128, 64), :].astype(jnp.float32))
        b1 = b0 + qoff(y, 64)

        exchange(
            [b1 + qoff(lax.rem(z + dz, 4), 16) for dz in (1, 2, 3)],
            [16] * 3,
            [RSZ, RSZ + 16, RSZ + 32],
            z_partners, [4, 5, 6])
        kr = pl.ds(b1 + qoff(z, 16), 16)
        out_ref[kr, :] = (
            out_ref[kr, :]
            + comm_ref[pl.ds(RSZ, 16), :].astype(jnp.float32)
            + comm_ref[pl.ds(RSZ + 16, 16), :].astype(jnp.float32)
            + comm_ref[pl.ds(RSZ + 32, 16), :].astype(jnp.float32))
        b2 = b1 + qoff(z, 16)

        exchange([b2] * 3, [16] * 3, [AGZ, AGZ + 16, AGZ + 32],
                 z_partners, [7, 8, 9])
        for i, dz in enumerate((1, 2, 3)):
            w = lax.rem(z - dz + 4, 4)
            out_ref[pl.ds(b1 + qoff(w, 16), 16), :] = (
                comm_ref[pl.ds(AGZ + i * 16, 16), :].astype(jnp.float32))

        exchange([b1] * 3, [64] * 3, [AGY, AGY + 64, AGY + 128],
                 y_partners, [10, 11, 12])
        for i, dy in enumerate((1, 2, 3)):
            w = lax.rem(y - dy + 4, 4)
            out_ref[pl.ds(b0 + qoff(w, 64), 64), :] = (
                comm_ref[pl.ds(AGY + i * 64, 64), :].astype(jnp.float32))

        exchange([b0], [256], [AGX], [x_partner], [13])
        out_ref[pl.ds((1 - xx) * 256, 256), :] = (
            comm_ref[pl.ds(AGX, 256), :].astype(jnp.float32))

        @functools.partial(pl.run_scoped,
                           second_barrier=pltpu.SemaphoreType.REGULAR)
        def _(second_barrier):
            for nbr in all_partners:
                pl.semaphore_signal(second_barrier, inc=1, device_id=(nbr,),
                                    device_id_type=pl.DeviceIdType.MESH)
            pl.semaphore_wait(second_barrier, len(all_partners))

    def body(x_ref, wq_ref, k_hbm, v_hbm, wo_ref, out_ref,
             comm_ref, stage_ref, buf_ref, k_loc, v_loc, kv_sems,
             send_sems, recv_sems):
        p = lax.axis_index("i")
        z = p // 8
        j = lax.rem(p, 8)
        y = j // 2
        xx = lax.rem(j + y, 2)

        def logical(x_, y_, z_):
            return 8 * z_ + 2 * y_ + lax.rem(x_ + y_, 2)

        bits = [xx, lax.rem(y, 2), y // 2, lax.rem(z, 2), z // 2]
        partners = [
            logical(1 - xx, y, z),
            logical(xx, y + 1 - 2 * bits[1], z),
            logical(xx, y + 2 - 4 * bits[2], z),
            logical(xx, y, z + 1 - 2 * bits[3]),
            logical(xx, y, z + 2 - 4 * bits[4]),
        ]

        h0 = p * H_local
        chunk = Skv // 2
        n_chunks = 2 * B * 2

        def chunk_src(t):
            src = k_hbm if t < 4 else v_hbm
            b, sh = (t % 4) // 2, t % 2
            return src.at[b, pl.ds(sh * chunk, chunk), :, :]

        kv_copies = {}
        if not _SKIP_KV:
            for t in (0, 1):
                kv_copies[t] = pltpu.make_async_copy(
                    chunk_src(t), buf_ref.at[t % 2], kv_sems.at[t % 2])
                kv_copies[t].start()

        xf = x_ref[...].reshape(rows, E).astype(jnp.bfloat16)
        wq = wq_ref[...].astype(jnp.bfloat16)
        q_all = jnp.dot(xf, wq, preferred_element_type=jnp.float32)

        qb = lax.broadcasted_iota(jnp.int32, (Sq, Skv), 0) // 64
        kb = lax.broadcasted_iota(jnp.int32, (Sq, Skv), 1) // 64
        mask = (qb == kb) | ((kb % 4) == (qb % 4))

        if _SKIP_ATTN:
            out_ref[...] = jnp.dot(xf, xf, preferred_element_type=jnp.float32)
            _comm(p, xx, y, z, out_ref, comm_ref, stage_ref,
                  send_sems, recv_sems)
            return

        hi = lax.broadcasted_iota(jnp.int32, (H_local, H_glob), 0)
        gi = lax.broadcasted_iota(jnp.int32, (H_local, H_glob), 1)
        S4 = (gi == h0 + hi).astype(jnp.bfloat16)

        if not _SKIP_KV:
            for t in range(n_chunks):
                kv_copies[t].wait()
                loc = k_loc if t < 4 else v_loc
                b, sh = (t % 4) // 2, t % 2
                v2d = buf_ref[t % 2].astype(jnp.bfloat16).reshape(
                    chunk * DH, H_glob)
                sl4 = jnp.dot(S4, v2d.T,
                              preferred_element_type=jnp.float32)
                sl3 = sl4.astype(jnp.bfloat16).reshape(H_local, chunk, DH)
                for hl in range(H_local):
                    loc[b, pl.ds(sh * chunk, chunk),
                        hl * DH:(hl + 1) * DH] = sl3[hl]
                if t + 2 < n_chunks:
                    kv_copies[t + 2] = pltpu.make_async_copy(
                        chunk_src(t + 2), buf_ref.at[t % 2],
                        kv_sems.at[t % 2])
                    kv_copies[t + 2].start()

        ctx_rows = []
        for b in range(B):
            ctx_heads = []
            for h in range(H_local):
                q = q_all[b * Sq:(b + 1) * Sq, h * DH:(h + 1) * DH]
                k = k_loc[b, :, h * DH:(h + 1) * DH]
                s = jnp.dot(q.astype(jnp.bfloat16), k.T,
                            preferred_element_type=jnp.float32) * 0.125
                s = jnp.where(mask, s, -1e9)
                s = s - jnp.max(s, axis=1, keepdims=True)
                e = jnp.exp(s)
                w = e / jnp.sum(e, axis=1, keepdims=True)
                v = v_loc[b, :, h * DH:(h + 1) * DH]
                ctx_heads.append(jnp.dot(w.astype(jnp.bfloat16), v,
                                         preferred_element_type=jnp.float32))
            ctx_rows.append(jnp.concatenate(ctx_heads, axis=1))
        ctx = jnp.concatenate(ctx_rows, axis=0)

        wo = wo_ref[...].astype(jnp.bfloat16)
        out_ref[...] = jnp.dot(ctx.astype(jnp.bfloat16), wo,
                               preferred_element_type=jnp.float32)

        _comm(p, xx, y, z, out_ref, comm_ref, stage_ref,
                  send_sems, recv_sems)

    out = pl.pallas_call(
        body,
        out_shape=jax.ShapeDtypeStruct((rows, E), jnp.float32),
        in_specs=[
            pl.BlockSpec(memory_space=pltpu.VMEM),
            pl.BlockSpec(memory_space=pltpu.VMEM),
            pl.BlockSpec(memory_space=pltpu.MemorySpace.HBM),
            pl.BlockSpec(memory_space=pltpu.MemorySpace.HBM),
            pl.BlockSpec(memory_space=pltpu.VMEM),
        ],
        out_specs=pl.BlockSpec(memory_space=pltpu.VMEM),
        scratch_shapes=[
            pltpu.VMEM((2 * comm_rows, E), jnp.bfloat16),
            pltpu.VMEM((rows // 2, E), jnp.bfloat16),
            pltpu.VMEM((2, Skv // 2, DH, H_glob), jnp.float32),
            pltpu.VMEM((B, Skv, H_local * DH), jnp.bfloat16),
            pltpu.VMEM((B, Skv, H_local * DH), jnp.bfloat16),
            pltpu.SemaphoreType.DMA((2,)),
            pltpu.SemaphoreType.DMA((14,)),
            pltpu.SemaphoreType.DMA((14,)),
        ],
        compiler_params=pltpu.CompilerParams(
            collective_id=None if _SKIP_COMM else 0),
    )(x, Wq, K2, V2, Wo)
    return out.reshape(B, Sq, E)
